# baseline (speedup 1.0000x reference)
import functools

import jax
import jax.numpy as jnp
from jax import lax
from jax.experimental import pallas as pl
from jax.experimental.pallas import tpu as pltpu
from jax.experimental.pallas import tpu_sc as plsc

EMBED_DIM = 64
NUM_BINS = 50
VOCAB = NUM_BINS + 3
B = 4096
G = 200

NC = 2
NS = 16
NW = NC * NS
UNITS = G * 8
PER_W = UNITS // NW
BLV = B // 16


def _sc_kernel(idx_hbm, norm_hbm, table_hbm, w_hbm, b_hbm, out_hbm,
               table_v, w_v, b_v, wsplat_v, idx_v, norm_v, out_v):
    wid = lax.axis_index("s") * NC + lax.axis_index("c")

    pltpu.sync_copy(table_hbm, table_v)
    pltpu.sync_copy(w_hbm, w_v)
    pltpu.sync_copy(b_hbm, b_v)

    def fold_b(i, _):
        for j in range(4):
            s = pl.ds(i * EMBED_DIM + j * 16, 16)
            table_v[s] = table_v[s] + b_v[pl.ds(j * 16, 16)]
        return 0
    lax.fori_loop(0, VOCAB, fold_b, 0)

    for j in range(4):
        wj = w_v[pl.ds(16 * j, 16)]
        for l in range(16):
            wsplat_v[pl.ds((16 * j + l) * 16, 16)] = jnp.broadcast_to(
                wj[l], (16,))

    def unit_body(k, _):
        u = wid * PER_W + k
        g = u % G
        dh = u // G
        pltpu.sync_copy(idx_hbm.at[g], idx_v)
        pltpu.sync_copy(norm_hbm.at[g], norm_v)

        dbase = dh * 8
        wds = tuple(wsplat_v[pl.ds((dbase + dl) * 16, 16)] for dl in range(8))
        dh8 = jnp.broadcast_to(dbase, (16,))

        @plsc.parallel_loop(0, BLV)
        def blv_body(i):
            iv = idx_v[pl.ds(i * 16, 16)]
            nv = norm_v[pl.ds(i * 16, 16)]
            base = iv * EMBED_DIM + dh8
            bh = i // 8
            bl = (i % 8) * 16
            for dl in range(8):
                row = plsc.load_gather(table_v, [base + dl])
                out_v[bh, dl, pl.ds(bl, 16)] = row + nv * wds[dl]

        pltpu.sync_copy(out_v, out_hbm.at[g, dh])
        return 0
    lax.fori_loop(0, PER_W, unit_body, 0)


@jax.jit
def _run(idx, norm, table, w, b):
    mesh = plsc.VectorSubcoreMesh(core_axis_name="c", subcore_axis_name="s")
    kern = functools.partial(
        pl.kernel,
        mesh=mesh,
        compiler_params=pltpu.CompilerParams(needs_layout_passes=False),
        out_type=jax.ShapeDtypeStruct((G, 8, 32, 8, 128), jnp.float32),
        scratch_types=[
            pltpu.VMEM((VOCAB * EMBED_DIM,), jnp.float32),
            pltpu.VMEM((EMBED_DIM,), jnp.float32),
            pltpu.VMEM((EMBED_DIM,), jnp.float32),
            pltpu.VMEM((EMBED_DIM * 16,), jnp.float32),
            pltpu.VMEM((B,), jnp.int32),
            pltpu.VMEM((B,), jnp.float32),
            pltpu.VMEM((32, 8, 128), jnp.float32),
        ],
    )(_sc_kernel)
    out5 = kern(idx, norm, table, w, b)
    return out5.transpose(2, 4, 0, 1, 3).reshape(B, G, EMBED_DIM)


def kernel(discrete_expression, normalized_expr, bin_table, W, b):
    idx = discrete_expression.astype(jnp.int32).T
    norm = normalized_expr.T
    table = bin_table.reshape(-1)
    w = W.reshape(-1)
    return _run(idx, norm, table, w, b)
